# baseline (speedup 1.0000x reference)
import functools

import jax
import jax.numpy as jnp
from jax import lax
from jax.experimental import pallas as pl
from jax.experimental.pallas import tpu as pltpu
from jax.experimental.pallas import tpu_sc as plsc

N = 10000
E = 320000
NC = 2
NS = 16
NW = NC * NS
EPW = E // NW
B = 125
NCH = EPW // B
SB = 80
NP = 10240
NPT = NP // NS
KTOP = 5000
DEGW = 16
RB = 512
CPT = NP // NW // SB

DIMS_P = [
    (128, 128), (128, 64), (64, 32), (32, 16), (16, 16),
    (16, 16), (16, 32), (32, 64), (64, 128), (128, 128),
]


def _mesh():
    return plsc.VectorSubcoreMesh(
        core_axis_name="c", subcore_axis_name="s",
        num_cores=NC, num_subcores=NS)


@functools.cache
def _deg_kernel():
    @functools.partial(
        pl.kernel,
        out_type=jax.ShapeDtypeStruct((NC, NP, DEGW), jnp.float32),
        mesh=_mesh(),
        compiler_params=pltpu.CompilerParams(use_tc_tiling_on_sc=False),
        scratch_types=[
            pltpu.VMEM((NCH, B), jnp.int32),
            pltpu.VMEM((B, DEGW), jnp.float32),
            pltpu.VMEM_SHARED((NP, DEGW), jnp.float32),
            pltpu.SemaphoreType.DMA,
        ],
    )
    def deg(dst_hbm, ones_hbm, zero_hbm, out_hbm, dst_v, ones_v, acc_sh, dsem):
        c = lax.axis_index("c")
        s = lax.axis_index("s")
        w = c * NS + s
        pltpu.sync_copy(zero_hbm.at[pl.ds(s * NPT, NPT)],
                        acc_sh.at[pl.ds(s * NPT, NPT)])
        pltpu.sync_copy(dst_hbm.at[w], dst_v)
        pltpu.sync_copy(ones_hbm, ones_v)
        plsc.subcore_barrier()

        def body(gi, carry):
            base = gi * 8
            for b in range(8):
                pltpu.async_copy(ones_v, acc_sh.at[dst_v.at[base + b]],
                                 dsem, add=True)
            for b in range(8):
                pltpu.make_async_copy(ones_v, acc_sh.at[dst_v.at[base + b]],
                                      dsem).wait()
            return carry

        lax.fori_loop(0, NCH // 8, body, 0)
        plsc.subcore_barrier()
        pltpu.sync_copy(acc_sh.at[pl.ds(s * NPT, NPT)],
                        out_hbm.at[c].at[pl.ds(s * NPT, NPT)])

    return deg


@functools.cache
def _prop_kernel(C):
    eb, nb = (100, 2) if C == 128 else (125, 4)
    nch = EPW // eb

    @functools.partial(
        pl.kernel,
        out_type=jax.ShapeDtypeStruct((NC, NP, C), jnp.float32),
        mesh=_mesh(),
        compiler_params=pltpu.CompilerParams(use_tc_tiling_on_sc=False),
        scratch_types=[
            pltpu.VMEM((nch, eb), jnp.int32),
            pltpu.VMEM((nch, eb), jnp.int32),
            pltpu.VMEM((nb, eb, C), jnp.float32),
            pltpu.VMEM_SHARED((NP, C), jnp.float32),
            pltpu.SemaphoreType.DMA((nb,)),
            pltpu.SemaphoreType.DMA((nb,)),
        ],
    )
    def prop(g_hbm, src_hbm, dst_hbm, zero_hbm, out_hbm,
             src_v, dst_v, rows_v, acc_sh, gsem, ssem):
        c = lax.axis_index("c")
        s = lax.axis_index("s")
        w = c * NS + s
        pltpu.sync_copy(zero_hbm.at[pl.ds(s * NPT, NPT)],
                        acc_sh.at[pl.ds(s * NPT, NPT)])
        pltpu.sync_copy(src_hbm.at[w], src_v)
        pltpu.sync_copy(dst_hbm.at[w], dst_v)
        plsc.subcore_barrier()

        def start_gather(j, b):
            pltpu.async_copy(g_hbm.at[src_v.at[j]], rows_v.at[b], gsem.at[b])

        def start_scatter(j, b):
            pltpu.make_async_copy(
                g_hbm.at[src_v.at[j]], rows_v.at[b], gsem.at[b]).wait()
            pltpu.async_copy(rows_v.at[b], acc_sh.at[dst_v.at[j]],
                             ssem.at[b], add=True)

        def wait_scatter(j, b):
            pltpu.make_async_copy(rows_v.at[b], acc_sh.at[dst_v.at[j]],
                                  ssem.at[b]).wait()

        for b in range(nb):
            start_gather(b, b)

        def group(gi, carry):
            base = gi * nb
            for b in range(nb):
                start_scatter(base + b, b)
            for b in range(nb):
                wait_scatter(base + b, b)
                start_gather(base + b + nb, b)
            return carry

        lax.fori_loop(0, nch // nb - 1, group, 0)
        base = nch - nb
        for b in range(nb):
            start_scatter(base + b, b)
        for b in range(nb):
            wait_scatter(base + b, b)
        plsc.subcore_barrier()
        pltpu.sync_copy(acc_sh.at[pl.ds(s * NPT, NPT)],
                        out_hbm.at[c].at[pl.ds(s * NPT, NPT)])

    return prop


@functools.cache
def _scatter_kernel():
    @functools.partial(
        pl.kernel,
        out_type=jax.ShapeDtypeStruct((NP, 128), jnp.float32),
        mesh=_mesh(),
        compiler_params=pltpu.CompilerParams(use_tc_tiling_on_sc=False),
        scratch_types=[
            pltpu.VMEM((CPT, SB), jnp.int32),
            pltpu.VMEM((SB, 128), jnp.float32),
            pltpu.SemaphoreType.DMA,
        ],
    )
    def scat(hg_hbm, rank_hbm, out_hbm, rank_v, rows_v, sem):
        c = lax.axis_index("c")
        s = lax.axis_index("s")
        w = c * NS + s
        pltpu.sync_copy(rank_hbm.at[w], rank_v)
        for j in range(CPT):
            base = (w * CPT + j) * SB
            pltpu.sync_copy(hg_hbm.at[pl.ds(base, SB)], rows_v)
            pltpu.async_copy(rows_v, out_hbm.at[rank_v.at[j]], sem).wait()

    return scat


def _tc_first(x, w, dega, degb):
    def body(x_ref, w_ref, da_ref, db_ref, g_ref, dinv_ref):
        deg = da_ref[:, 0:1] + db_ref[:, 0:1] + 1.0
        dinv = 1.0 / jnp.sqrt(deg)
        dinv_ref[...] = dinv
        g_ref[...] = jnp.dot(x_ref[...], w_ref[...],
                             preferred_element_type=jnp.float32) * dinv

    return pl.pallas_call(
        body,
        out_shape=[
            jax.ShapeDtypeStruct((NP, 128), jnp.float32),
            jax.ShapeDtypeStruct((NP, 1), jnp.float32),
        ],
    )(x, w, dega, degb)


def _tc_mid(aa, ab, gp, dinv, w, b):
    cout = w.shape[1]

    def body(aa_ref, ab_ref, gp_ref, dinv_ref, w_ref, b_ref, g_ref):
        d = dinv_ref[...]
        h = d * (aa_ref[...] + ab_ref[...] + gp_ref[...]) + b_ref[...]
        h = jnp.maximum(h, 0.0)
        g_ref[...] = d * jnp.dot(h, w_ref[...],
                                 preferred_element_type=jnp.float32)

    return pl.pallas_call(
        body,
        out_shape=jax.ShapeDtypeStruct((NP, cout), jnp.float32),
    )(aa, ab, gp, dinv, w, b)


def _tc_final(aa, ab, gp, dinv, b, p):
    def body(aa_ref, ab_ref, gp_ref, dinv_ref, b_ref, p_ref, hg_ref, sc_ref):
        d = dinv_ref[...]
        h = d * (aa_ref[...] + ab_ref[...] + gp_ref[...]) + b_ref[...]
        pv = p_ref[...]
        pn = pv / jnp.sqrt(jnp.sum(pv * pv))
        score = jnp.sum(h * pn, axis=1, keepdims=True)
        row = lax.broadcasted_iota(jnp.int32, (NP, 1), 0)
        score = jnp.where(row < N, score, -jnp.inf)
        sc_ref[...] = score
        hg_ref[...] = h * jnp.tanh(score)

    return pl.pallas_call(
        body,
        out_shape=[
            jax.ShapeDtypeStruct((NP, 128), jnp.float32),
            jax.ShapeDtypeStruct((NP, 1), jnp.float32),
        ],
    )(aa, ab, gp, dinv, b, p)


def _rank(scol, srow):
    def body(sc_ref, sr_ref, r_ref):
        i = pl.program_id(0)
        j = pl.program_id(1)

        @pl.when(j == 0)
        def _():
            r_ref[...] = jnp.zeros_like(r_ref)

        si = sc_ref[...]
        sj = sr_ref[...]

        @pl.when(j < i)
        def _():
            r_ref[...] += jnp.sum((sj >= si).astype(jnp.int32),
                                  axis=1, keepdims=True)

        @pl.when(j > i)
        def _():
            r_ref[...] += jnp.sum((sj > si).astype(jnp.int32),
                                  axis=1, keepdims=True)

        @pl.when(j == i)
        def _():
            tri = (lax.broadcasted_iota(jnp.int32, (RB, RB), 1)
                   < lax.broadcasted_iota(jnp.int32, (RB, RB), 0))
            cnt = (sj > si) | ((sj == si) & tri)
            r_ref[...] += jnp.sum(cnt.astype(jnp.int32),
                                  axis=1, keepdims=True)

    return pl.pallas_call(
        body,
        grid=(NP // RB, NP // RB),
        in_specs=[
            pl.BlockSpec((RB, 1), lambda i, j: (i, 0)),
            pl.BlockSpec((1, RB), lambda i, j: (0, j)),
        ],
        out_specs=pl.BlockSpec((RB, 1), lambda i, j: (i, 0)),
        out_shape=jax.ShapeDtypeStruct((NP, 1), jnp.int32),
    )(scol, srow)


def kernel(x, params, edge_index):
    edges = {
        eb: (edge_index[0].reshape(NW, EPW // eb, eb),
             edge_index[1].reshape(NW, EPW // eb, eb))
        for eb in (100, 125)
    }
    dst = edges[B][1]

    ws, bs = [], []
    for i, (fip, fop) in enumerate(DIMS_P):
        wi = params[f"W{i + 1}"]
        bi = params[f"b{i + 1}"]
        wi = jnp.pad(wi, ((0, fip - wi.shape[0]), (0, fop - wi.shape[1])))
        bi = jnp.pad(bi, (0, fop - bi.shape[0])).reshape(1, fop)
        ws.append(wi)
        bs.append(bi)

    xp = jnp.pad(x, ((0, NP - N), (0, 0)))
    ones = jnp.ones((B, DEGW), jnp.float32)
    zeros = {c: jnp.zeros((NP, c), jnp.float32) for c in (128, 64, 32, 16)}

    degp = _deg_kernel()(dst, ones, zeros[DEGW])
    g, dinv = _tc_first(xp, ws[0], degp[0], degp[1])
    for l in range(1, 10):
        cin = ws[l].shape[0]
        s2, d2 = edges[100 if cin == 128 else 125]
        acc = _prop_kernel(cin)(g, s2, d2, zeros[cin])
        g = _tc_mid(acc[0], acc[1], g, dinv, ws[l], bs[l - 1])
    accl = _prop_kernel(128)(g, edges[100][0], edges[100][1], zeros[128])
    hg, score = _tc_final(accl[0], accl[1], g, dinv, bs[9],
                          params["p"].reshape(1, 128))

    rank = _rank(score, score.reshape(1, NP))
    outf = _scatter_kernel()(hg, rank.reshape(NW, CPT, SB))
    return outf[:KTOP]


# speedup vs baseline: 1.1127x; 1.1127x over previous
import functools

import jax
import jax.numpy as jnp
from jax import lax
from jax.experimental import pallas as pl
from jax.experimental.pallas import tpu as pltpu
from jax.experimental.pallas import tpu_sc as plsc

N = 10000
E = 320000
NC = 2
NS = 16
NW = NC * NS
EPW = E // NW
B = 125
NCH = EPW // B
SB = 80
NP = 10240
NPT = NP // NS
KTOP = 5000
DEGW = 16
RB = 512
CPT = NP // NW // SB

DIMS_P = [
    (128, 128), (128, 64), (64, 32), (32, 16), (16, 16),
    (16, 16), (16, 32), (32, 64), (64, 128), (128, 128),
]


def _mesh():
    return plsc.VectorSubcoreMesh(
        core_axis_name="c", subcore_axis_name="s",
        num_cores=NC, num_subcores=NS)


@functools.cache
def _deg_kernel():
    @functools.partial(
        pl.kernel,
        out_type=jax.ShapeDtypeStruct((NC, NP, DEGW), jnp.float32),
        mesh=_mesh(),
        compiler_params=pltpu.CompilerParams(use_tc_tiling_on_sc=False),
        scratch_types=[
            pltpu.VMEM((NCH, B), jnp.int32),
            pltpu.VMEM((B, DEGW), jnp.float32),
            pltpu.VMEM_SHARED((NP, DEGW), jnp.float32),
            pltpu.SemaphoreType.DMA,
        ],
    )
    def deg(dst_hbm, ones_hbm, zero_hbm, out_hbm, dst_v, ones_v, acc_sh, dsem):
        c = lax.axis_index("c")
        s = lax.axis_index("s")
        w = c * NS + s
        pltpu.sync_copy(zero_hbm.at[pl.ds(s * NPT, NPT)],
                        acc_sh.at[pl.ds(s * NPT, NPT)])
        pltpu.sync_copy(dst_hbm.at[w], dst_v)
        pltpu.sync_copy(ones_hbm, ones_v)
        plsc.subcore_barrier()

        def body(gi, carry):
            base = gi * 8
            for b in range(8):
                pltpu.async_copy(ones_v, acc_sh.at[dst_v.at[base + b]],
                                 dsem, add=True)
            for b in range(8):
                pltpu.make_async_copy(ones_v, acc_sh.at[dst_v.at[base + b]],
                                      dsem).wait()
            return carry

        lax.fori_loop(0, NCH // 8, body, 0)
        plsc.subcore_barrier()
        pltpu.sync_copy(acc_sh.at[pl.ds(s * NPT, NPT)],
                        out_hbm.at[c].at[pl.ds(s * NPT, NPT)])

    return deg


@functools.cache
def _prop_kernel(C):
    eb, nb = (100, 2) if C == 128 else (125, 4)
    nch = EPW // eb

    @functools.partial(
        pl.kernel,
        out_type=jax.ShapeDtypeStruct((NC, NP, C), jnp.float32),
        mesh=_mesh(),
        compiler_params=pltpu.CompilerParams(use_tc_tiling_on_sc=False),
        scratch_types=[
            pltpu.VMEM((nch, eb), jnp.int32),
            pltpu.VMEM((nch, eb), jnp.int32),
            pltpu.VMEM((nb, eb, C), jnp.float32),
            pltpu.VMEM_SHARED((NP, C), jnp.float32),
            pltpu.SemaphoreType.DMA((nb,)),
        ],
    )
    def prop(g_hbm, src_hbm, dst_hbm, zero_hbm, out_hbm,
             src_v, dst_v, rows_v, acc_sh, gsem):
        c = lax.axis_index("c")
        s = lax.axis_index("s")
        w = c * NS + s
        pltpu.sync_copy(zero_hbm.at[pl.ds(s * NPT, NPT)],
                        acc_sh.at[pl.ds(s * NPT, NPT)])
        pltpu.sync_copy(src_hbm.at[w], src_v)
        pltpu.sync_copy(dst_hbm.at[w], dst_v)
        plsc.subcore_barrier()

        def start_gather(j, b):
            pltpu.async_copy(g_hbm.at[src_v.at[j]], rows_v.at[b], gsem.at[b])

        def finish_chunk(j, b):
            pltpu.make_async_copy(
                g_hbm.at[src_v.at[j]], rows_v.at[b], gsem.at[b]).wait()
            pltpu.sync_copy(rows_v.at[b], acc_sh.at[dst_v.at[j]], add=True)

        for b in range(nb):
            start_gather(b, b)

        def group(gi, carry):
            base = gi * nb
            for b in range(nb):
                finish_chunk(base + b, b)
                start_gather(base + b + nb, b)
            return carry

        lax.fori_loop(0, nch // nb - 1, group, 0)
        for b in range(nb):
            finish_chunk(nch - nb + b, b)
        plsc.subcore_barrier()
        pltpu.sync_copy(acc_sh.at[pl.ds(s * NPT, NPT)],
                        out_hbm.at[c].at[pl.ds(s * NPT, NPT)])

    return prop


@functools.cache
def _scatter_kernel():
    @functools.partial(
        pl.kernel,
        out_type=jax.ShapeDtypeStruct((NP, 128), jnp.float32),
        mesh=_mesh(),
        compiler_params=pltpu.CompilerParams(use_tc_tiling_on_sc=False),
        scratch_types=[
            pltpu.VMEM((CPT, SB), jnp.int32),
            pltpu.VMEM((SB, 128), jnp.float32),
            pltpu.SemaphoreType.DMA,
        ],
    )
    def scat(hg_hbm, rank_hbm, out_hbm, rank_v, rows_v, sem):
        c = lax.axis_index("c")
        s = lax.axis_index("s")
        w = c * NS + s
        pltpu.sync_copy(rank_hbm.at[w], rank_v)
        for j in range(CPT):
            base = (w * CPT + j) * SB
            pltpu.sync_copy(hg_hbm.at[pl.ds(base, SB)], rows_v)
            pltpu.async_copy(rows_v, out_hbm.at[rank_v.at[j]], sem).wait()

    return scat


def _tc_first(x, w, dega, degb):
    def body(x_ref, w_ref, da_ref, db_ref, g_ref, dinv_ref):
        deg = da_ref[:, 0:1] + db_ref[:, 0:1] + 1.0
        dinv = 1.0 / jnp.sqrt(deg)
        dinv_ref[...] = dinv
        g_ref[...] = jnp.dot(x_ref[...], w_ref[...],
                             preferred_element_type=jnp.float32) * dinv

    return pl.pallas_call(
        body,
        out_shape=[
            jax.ShapeDtypeStruct((NP, 128), jnp.float32),
            jax.ShapeDtypeStruct((NP, 1), jnp.float32),
        ],
    )(x, w, dega, degb)


def _tc_mid(aa, ab, gp, dinv, w, b):
    cout = w.shape[1]

    def body(aa_ref, ab_ref, gp_ref, dinv_ref, w_ref, b_ref, g_ref):
        d = dinv_ref[...]
        h = d * (aa_ref[...] + ab_ref[...] + gp_ref[...]) + b_ref[...]
        h = jnp.maximum(h, 0.0)
        g_ref[...] = d * jnp.dot(h, w_ref[...],
                                 preferred_element_type=jnp.float32)

    return pl.pallas_call(
        body,
        out_shape=jax.ShapeDtypeStruct((NP, cout), jnp.float32),
    )(aa, ab, gp, dinv, w, b)


def _tc_final(aa, ab, gp, dinv, b, p):
    def body(aa_ref, ab_ref, gp_ref, dinv_ref, b_ref, p_ref, hg_ref, sc_ref):
        d = dinv_ref[...]
        h = d * (aa_ref[...] + ab_ref[...] + gp_ref[...]) + b_ref[...]
        pv = p_ref[...]
        pn = pv / jnp.sqrt(jnp.sum(pv * pv))
        score = jnp.sum(h * pn, axis=1, keepdims=True)
        row = lax.broadcasted_iota(jnp.int32, (NP, 1), 0)
        score = jnp.where(row < N, score, -jnp.inf)
        sc_ref[...] = score
        hg_ref[...] = h * jnp.tanh(score)

    return pl.pallas_call(
        body,
        out_shape=[
            jax.ShapeDtypeStruct((NP, 128), jnp.float32),
            jax.ShapeDtypeStruct((NP, 1), jnp.float32),
        ],
    )(aa, ab, gp, dinv, b, p)


def _rank(scol, srow):
    def body(sc_ref, sr_ref, r_ref):
        i = pl.program_id(0)
        j = pl.program_id(1)

        @pl.when(j == 0)
        def _():
            r_ref[...] = jnp.zeros_like(r_ref)

        si = sc_ref[...]
        sj = sr_ref[...]
        onev = jnp.ones((RB, 1), jnp.float32)

        def accum(cmp):
            r_ref[...] += jnp.dot(cmp.astype(jnp.float32), onev,
                                  preferred_element_type=jnp.float32)

        @pl.when(j < i)
        def _():
            accum(sj >= si)

        @pl.when(j > i)
        def _():
            accum(sj > si)

        @pl.when(j == i)
        def _():
            tri = (lax.broadcasted_iota(jnp.int32, (RB, RB), 1)
                   < lax.broadcasted_iota(jnp.int32, (RB, RB), 0))
            accum((sj > si) | ((sj == si) & tri))

    return pl.pallas_call(
        body,
        grid=(NP // RB, NP // RB),
        in_specs=[
            pl.BlockSpec((RB, 1), lambda i, j: (i, 0)),
            pl.BlockSpec((1, RB), lambda i, j: (0, j)),
        ],
        out_specs=pl.BlockSpec((RB, 1), lambda i, j: (i, 0)),
        out_shape=jax.ShapeDtypeStruct((NP, 1), jnp.float32),
    )(scol, srow)


def kernel(x, params, edge_index):
    edges = {
        eb: (edge_index[0].reshape(NW, EPW // eb, eb),
             edge_index[1].reshape(NW, EPW // eb, eb))
        for eb in (100, 125)
    }
    dst = edges[B][1]

    ws, bs = [], []
    for i, (fip, fop) in enumerate(DIMS_P):
        wi = params[f"W{i + 1}"]
        bi = params[f"b{i + 1}"]
        wi = jnp.pad(wi, ((0, fip - wi.shape[0]), (0, fop - wi.shape[1])))
        bi = jnp.pad(bi, (0, fop - bi.shape[0])).reshape(1, fop)
        ws.append(wi)
        bs.append(bi)

    xp = jnp.pad(x, ((0, NP - N), (0, 0)))
    ones = jnp.ones((B, DEGW), jnp.float32)
    zeros = {c: jnp.zeros((NP, c), jnp.float32) for c in (128, 64, 32, 16)}

    degp = _deg_kernel()(dst, ones, zeros[DEGW])
    g, dinv = _tc_first(xp, ws[0], degp[0], degp[1])
    for l in range(1, 10):
        cin = ws[l].shape[0]
        s2, d2 = edges[100 if cin == 128 else 125]
        acc = _prop_kernel(cin)(g, s2, d2, zeros[cin])
        g = _tc_mid(acc[0], acc[1], g, dinv, ws[l], bs[l - 1])
    accl = _prop_kernel(128)(g, edges[100][0], edges[100][1], zeros[128])
    hg, score = _tc_final(accl[0], accl[1], g, dinv, bs[9],
                          params["p"].reshape(1, 128))

    rank = _rank(score, score.reshape(1, NP)).astype(jnp.int32)
    outf = _scatter_kernel()(hg, rank.reshape(NW, CPT, SB))
    return outf[:KTOP]
